# baseline (speedup 1.0000x reference)
import functools

import jax
import jax.numpy as jnp
from jax import lax
from jax.experimental import pallas as pl
from jax.experimental.pallas import tpu as pltpu
from jax.experimental.pallas import tpu_sc as plsc

VOCAB = 256
N_BLOCKS = 4
BLOCK_ROWS = 64
DIM = 64
B, L = 1024, 20
N_TOK = B * L

_info = plsc.get_sparse_core_info()
_NC, _NS, _LANES = _info.num_cores, _info.num_subcores, _info.num_lanes
_NW = _NC * _NS
_TOK_PER_W = N_TOK // _NW


def _make_sc_kernel():
    mesh = plsc.VectorSubcoreMesh(core_axis_name="c", subcore_axis_name="s")

    @functools.partial(
        pl.kernel,
        mesh=mesh,
        out_type=jax.ShapeDtypeStruct((N_TOK // 2, 2 * DIM), jnp.float32),
        compiler_params=pltpu.CompilerParams(use_tc_tiling_on_sc=False),
        scratch_types=[
            pltpu.VMEM((_TOK_PER_W,), jnp.int32),
            pltpu.VMEM((_TOK_PER_W, DIM), jnp.float32),
            pltpu.SemaphoreType.DMA,
        ],
    )
    def sc_kernel(src_hbm, table_hbm, out_hbm, idx_v, rows_v, sem):
        wid = lax.axis_index("s") * _NC + lax.axis_index("c")
        base = wid * _TOK_PER_W
        pltpu.sync_copy(src_hbm.at[pl.ds(base, _TOK_PER_W)], idx_v)
        pltpu.async_copy(table_hbm.at[idx_v], rows_v, sem).wait()
        rbase = (wid // 2) * _TOK_PER_W
        col = (wid % 2) * DIM
        pltpu.sync_copy(
            rows_v,
            out_hbm.at[pl.ds(rbase, _TOK_PER_W), pl.ds(col, DIM)])

    return sc_kernel


_sc_kernel = _make_sc_kernel()


def kernel(src, block_assign, local_assign, W0, W1, W2, W3):
    table = jnp.concatenate([W0, W1, W2, W3], axis=0)
    row_map = block_assign * BLOCK_ROWS + local_assign
    table = table.at[row_map].get(mode="promise_in_bounds", unique_indices=True)
    flat_src = src.reshape(N_TOK)
    src_perm = flat_src.reshape(_NW // 2, _TOK_PER_W, 2)
    src_perm = src_perm.transpose(0, 2, 1).reshape(N_TOK)
    out = _sc_kernel(src_perm, table)
    return out.reshape(B, L, DIM)


# speedup vs baseline: 1.0719x; 1.0719x over previous
import functools

import jax
import jax.numpy as jnp
from jax import lax
from jax.experimental import pallas as pl
from jax.experimental.pallas import tpu as pltpu
from jax.experimental.pallas import tpu_sc as plsc

VOCAB = 256
N_BLOCKS = 4
BLOCK_ROWS = 64
DIM = 64
B, L = 1024, 20
N_TOK = B * L

_info = plsc.get_sparse_core_info()
_NC, _NS, _LANES = _info.num_cores, _info.num_subcores, _info.num_lanes
_NW = _NC * _NS
_TOK_PER_W = N_TOK // _NW
_HALF = _TOK_PER_W // 2


def _make_sc_kernel():
    mesh = plsc.VectorSubcoreMesh(core_axis_name="c", subcore_axis_name="s")

    @functools.partial(
        pl.kernel,
        mesh=mesh,
        out_type=jax.ShapeDtypeStruct((N_TOK, DIM), jnp.float32),
        compiler_params=pltpu.CompilerParams(use_tc_tiling_on_sc=False),
        scratch_types=[
            pltpu.VMEM((_TOK_PER_W,), jnp.int32),
            pltpu.VMEM((_TOK_PER_W, DIM), jnp.float32),
            pltpu.SemaphoreType.DMA,
            pltpu.SemaphoreType.DMA,
            pltpu.SemaphoreType.DMA,
        ],
    )
    def sc_kernel(src_hbm, table_hbm, out_hbm, idx_v, rows_v, sa, sb, sw):
        wid = lax.axis_index("s") * _NC + lax.axis_index("c")
        base = wid * _TOK_PER_W
        pltpu.sync_copy(src_hbm.at[pl.ds(base, _TOK_PER_W)], idx_v)
        ga = pltpu.async_copy(
            table_hbm.at[idx_v.at[pl.ds(0, _HALF)]],
            rows_v.at[pl.ds(0, _HALF)], sa)
        gb = pltpu.async_copy(
            table_hbm.at[idx_v.at[pl.ds(_HALF, _HALF)]],
            rows_v.at[pl.ds(_HALF, _HALF)], sb)
        ga.wait()
        wa = pltpu.async_copy(
            rows_v.at[pl.ds(0, _HALF)],
            out_hbm.at[pl.ds(base, _HALF)], sw)
        gb.wait()
        wb = pltpu.async_copy(
            rows_v.at[pl.ds(_HALF, _HALF)],
            out_hbm.at[pl.ds(base + _HALF, _HALF)], sw)
        wa.wait()
        wb.wait()

    return sc_kernel


_sc_kernel = _make_sc_kernel()


def kernel(src, block_assign, local_assign, W0, W1, W2, W3):
    table = jnp.concatenate([W0, W1, W2, W3], axis=0)
    row_map = block_assign * BLOCK_ROWS + local_assign
    table = table.at[row_map].get(mode="promise_in_bounds", unique_indices=True)
    flat_src = src.reshape(N_TOK)
    out = _sc_kernel(flat_src, table)
    return out.reshape(B, L, DIM)


# speedup vs baseline: 1.0810x; 1.0085x over previous
import functools

import jax
import jax.numpy as jnp
from jax import lax
from jax.experimental import pallas as pl
from jax.experimental.pallas import tpu as pltpu
from jax.experimental.pallas import tpu_sc as plsc

VOCAB = 256
N_BLOCKS = 4
BLOCK_ROWS = 64
DIM = 64
B, L = 1024, 20
N_TOK = B * L

_info = plsc.get_sparse_core_info()
_NC, _NS, _LANES = _info.num_cores, _info.num_subcores, _info.num_lanes
_NW = _NC * _NS
_TOK_PER_W = N_TOK // _NW


def _make_sc_kernel():
    mesh = plsc.VectorSubcoreMesh(core_axis_name="c", subcore_axis_name="s")

    @functools.partial(
        pl.kernel,
        mesh=mesh,
        out_type=jax.ShapeDtypeStruct((B, L, DIM), jnp.float32),
        compiler_params=pltpu.CompilerParams(use_tc_tiling_on_sc=False),
        scratch_types=[
            pltpu.VMEM((_TOK_PER_W,), jnp.int32),
            pltpu.VMEM((_TOK_PER_W, DIM), jnp.float32),
            pltpu.SemaphoreType.DMA,
        ],
    )
    def sc_kernel(src_hbm, table_hbm, out_hbm, idx_v, rows_v, sem):
        wid = lax.axis_index("s") * _NC + lax.axis_index("c")
        base = wid * _TOK_PER_W
        pltpu.sync_copy(src_hbm.at[pl.ds(base, _TOK_PER_W)], idx_v)
        pltpu.async_copy(table_hbm.at[idx_v], rows_v, sem).wait()
        rows_per_w = B // _NW
        bbase = wid * rows_per_w
        copies = [
            pltpu.async_copy(rows_v.at[pl.ds(k * L, L)], out_hbm.at[bbase + k], sem)
            for k in range(rows_per_w)
        ]
        for c in copies:
            c.wait()

    return sc_kernel


_sc_kernel = _make_sc_kernel()


def kernel(src, block_assign, local_assign, W0, W1, W2, W3):
    table = jnp.concatenate([W0, W1, W2, W3], axis=0)
    row_map = block_assign * BLOCK_ROWS + local_assign
    table = table.at[row_map].get(mode="promise_in_bounds", unique_indices=True)
    flat_src = src.reshape(N_TOK)
    return _sc_kernel(flat_src, table)
